# Initial kernel scaffold; baseline (speedup 1.0000x reference)
import jax
import jax.numpy as jnp
from jax.experimental import pallas as pl


def kernel(spatial_features, temporal_features, external_features, edges, params):
    raise NotImplementedError("write your pallas kernel here")


# speedup vs baseline: 12.0055x; 12.0055x over previous
import functools

import jax
import jax.numpy as jnp
from jax import lax
from jax.experimental import pallas as pl
from jax.experimental.pallas import tpu as pltpu
from jax.experimental.pallas import tpu_sc as plsc

N = 100000
E = 1600000
NP = 102400
DUMMY = N
CHUNK = 128
EP = 1605632
NCHUNKS = EP // CHUNK
NWORK = 32
CPW = NCHUNKS // NWORK
IBLK = 8
NBLOCKS = CPW // IBLK
NBUF = 4
ROWS_PER_TILE = NP // 16
ZROWS = 800
F32 = jnp.float32


@functools.partial(
    pl.kernel,
    out_type=jax.ShapeDtypeStruct((2 * NP, 16), F32),
    mesh=plsc.VectorSubcoreMesh(core_axis_name="c", subcore_axis_name="s"),
    compiler_params=pltpu.CompilerParams(use_tc_tiling_on_sc=False),
    scratch_types=[
        pltpu.VMEM((IBLK, CHUNK), jnp.int32),
        pltpu.VMEM((IBLK, CHUNK), jnp.int32),
        pltpu.VMEM((ZROWS, 16), F32),
        pltpu.VMEM((CHUNK, 16), F32),
        pltpu.VMEM((CHUNK, 16), F32),
        pltpu.VMEM((CHUNK, 16), F32),
        pltpu.VMEM((CHUNK, 16), F32),
        pltpu.VMEM_SHARED((NP, 16), F32),
        pltpu.SemaphoreType.DMA,
        pltpu.SemaphoreType.DMA,
        pltpu.SemaphoreType.DMA,
        pltpu.SemaphoreType.DMA,
    ],
)
def _edge_pass(src_hbm, dst_hbm, z_hbm, zero_hbm, out_hbm,
               srcb, dstb, zbuf, r0, r1, r2, r3, acc, s0, s1, s2, s3):
    c = lax.axis_index("c")
    s = lax.axis_index("s")
    w = s * 2 + c
    rows = [r0, r1, r2, r3]
    sems = [s0, s1, s2, s3]

    pltpu.sync_copy(zero_hbm, zbuf)
    row0 = s * ROWS_PER_TILE
    for i in range(ROWS_PER_TILE // ZROWS):
        pltpu.sync_copy(zbuf, acc.at[pl.ds(row0 + i * ZROWS, ZROWS)])
    plsc.subcore_barrier()

    chunk0 = w * CPW

    def blk(bi, carry):
        base = chunk0 + bi * IBLK
        pltpu.sync_copy(src_hbm.at[pl.ds(base, IBLK)], srcb)
        pltpu.sync_copy(dst_hbm.at[pl.ds(base, IBLK)], dstb)
        handles = []
        for j in range(IBLK):
            handles.append(
                pltpu.async_copy(z_hbm.at[srcb.at[j]], rows[j % NBUF],
                                 sems[j % NBUF]))
            if j >= NBUF - 1:
                k = j - (NBUF - 1)
                handles[k].wait()
                pltpu.sync_copy(rows[k % NBUF], acc.at[dstb.at[k]], add=True)
        for k in range(IBLK - NBUF + 1, IBLK):
            handles[k].wait()
            pltpu.sync_copy(rows[k % NBUF], acc.at[dstb.at[k]], add=True)
        return carry

    lax.fori_loop(0, NBLOCKS, blk, 0)
    plsc.subcore_barrier()

    pltpu.sync_copy(acc.at[pl.ds(row0, ROWS_PER_TILE)],
                    out_hbm.at[pl.ds(c * NP + row0, ROWS_PER_TILE)])


def kernel(spatial_features, temporal_features, external_features, edges, params):
    src = edges[0]
    dst = edges[1]
    pad = EP - E
    src2d = jnp.concatenate(
        [src, jnp.zeros((pad,), jnp.int32)]).reshape(NCHUNKS, CHUNK)
    dst2d = jnp.concatenate(
        [dst, jnp.full((pad,), DUMMY, jnp.int32)]).reshape(NCHUNKS, CHUNK)
    zeros_hbm = jnp.zeros((ZROWS, 16), F32)

    def seg16(z16):
        zp = jnp.concatenate([z16, jnp.zeros((NP - N, 16), F32)], axis=0)
        o = _edge_pass(src2d, dst2d, zp, zeros_hbm)
        return o[:N] + o[NP:NP + N]

    def agg(z):
        wd = z.shape[1]
        ns = -(-wd // 16)
        zq = jnp.pad(z, ((0, 0), (0, ns * 16 - wd)))
        parts = [seg16(zq[:, 16 * k:16 * (k + 1)]) for k in range(ns)]
        return jnp.concatenate(parts, axis=1)[:, :wd]

    deg = seg16(jnp.ones((N, 16), F32))[:, 0] + 1.0
    dinv = lax.rsqrt(deg)
    dcol = dinv[:, None]

    def cgn(x, layers):
        h = x
        for (W, b) in layers:
            z = (h @ W) * dcol
            sagg = agg(z)
            h = jax.nn.relu(dcol * (sagg + z) + b)
        return h

    def fc(x, layers):
        for W, b in layers[:-1]:
            x = jax.nn.relu(x @ W + b)
        W, b = layers[-1]
        return x @ W + b

    p = params
    h = fc(spatial_features, p['spatial_fc'])
    h = h + cgn(h, p['gcn0'])
    h = h + cgn(h, p['gcn1'])
    s_out = cgn(h, p['gcn2'])

    ht = temporal_features[:, 0, :]
    for bp in p['st']:
        sp = cgn(ht, bp['gcn'])
        temp = sp @ bp['conv_w'][:, :, 1].T + bp['conv_b']
        ht = jnp.concatenate([ht, temp], axis=1)
    t_out = jnp.stack([ht[:, 0:12].mean(axis=1), ht[:, 12:24].mean(axis=1)],
                      axis=1)

    e_out = fc(external_features, p['ext_fc'])
    feats = jnp.concatenate([s_out, t_out, e_out], axis=-1)
    W, b = p['out'][0]
    return jax.nn.relu(feats) @ W + b


# speedup vs baseline: 12.9997x; 1.0828x over previous
import functools

import jax
import jax.numpy as jnp
from jax import lax
from jax.experimental import pallas as pl
from jax.experimental.pallas import tpu as pltpu
from jax.experimental.pallas import tpu_sc as plsc

N = 100000
E = 1600000
NP = 102400
DUMMY = N
CHUNK = 128
EP = 1605632
NCHUNKS = EP // CHUNK
NWORK = 32
CPW = NCHUNKS // NWORK
IBLK = 14
NBLOCKS = CPW // IBLK
NBUF = 8
DEPTH = 4
ROWS_PER_TILE = NP // 16
ZROWS = 400
F32 = jnp.float32


@functools.partial(
    pl.kernel,
    out_type=jax.ShapeDtypeStruct((2 * NP, 16), F32),
    mesh=plsc.VectorSubcoreMesh(core_axis_name="c", subcore_axis_name="s"),
    compiler_params=pltpu.CompilerParams(use_tc_tiling_on_sc=False),
    scratch_types=[
        pltpu.VMEM((IBLK, CHUNK), jnp.int32),
        pltpu.VMEM((IBLK, CHUNK), jnp.int32),
        pltpu.VMEM((ZROWS, 16), F32),
        *[pltpu.VMEM((CHUNK, 16), F32) for _ in range(NBUF)],
        pltpu.VMEM_SHARED((NP, 16), F32),
        *[pltpu.SemaphoreType.DMA for _ in range(2 * NBUF)],
    ],
)
def _edge_pass(src_hbm, dst_hbm, z_hbm, zero_hbm, out_hbm,
               srcb, dstb, zbuf, *rest):
    rows = list(rest[:NBUF])
    acc = rest[NBUF]
    gsems = list(rest[NBUF + 1:NBUF + 1 + NBUF])
    ssems = list(rest[NBUF + 1 + NBUF:])
    c = lax.axis_index("c")
    s = lax.axis_index("s")
    w = s * 2 + c

    pltpu.sync_copy(zero_hbm, zbuf)
    row0 = s * ROWS_PER_TILE
    for i in range(ROWS_PER_TILE // ZROWS):
        pltpu.sync_copy(zbuf, acc.at[pl.ds(row0 + i * ZROWS, ZROWS)])
    plsc.subcore_barrier()

    chunk0 = w * CPW

    def blk(bi, carry):
        base = chunk0 + bi * IBLK
        pltpu.sync_copy(src_hbm.at[pl.ds(base, IBLK)], srcb)
        pltpu.sync_copy(dst_hbm.at[pl.ds(base, IBLK)], dstb)
        gh = [None] * IBLK
        sh = [None] * IBLK

        def issue_scatter(k):
            gh[k].wait()
            sh[k] = pltpu.async_copy(rows[k % NBUF], acc.at[dstb.at[k]],
                                     ssems[k % NBUF], add=True)

        for j in range(IBLK):
            if j >= NBUF:
                sh[j - NBUF].wait()
            gh[j] = pltpu.async_copy(z_hbm.at[srcb.at[j]], rows[j % NBUF],
                                     gsems[j % NBUF])
            if j >= DEPTH:
                issue_scatter(j - DEPTH)
        for k in range(IBLK - DEPTH, IBLK):
            issue_scatter(k)
        for k in range(max(0, IBLK - NBUF), IBLK):
            sh[k].wait()
        return carry

    lax.fori_loop(0, NBLOCKS, blk, 0)
    plsc.subcore_barrier()

    pltpu.sync_copy(acc.at[pl.ds(row0, ROWS_PER_TILE)],
                    out_hbm.at[pl.ds(c * NP + row0, ROWS_PER_TILE)])


def kernel(spatial_features, temporal_features, external_features, edges, params):
    src = edges[0]
    dst = edges[1]
    pad = EP - E
    src2d = jnp.concatenate(
        [src, jnp.zeros((pad,), jnp.int32)]).reshape(NCHUNKS, CHUNK)
    dst2d = jnp.concatenate(
        [dst, jnp.full((pad,), DUMMY, jnp.int32)]).reshape(NCHUNKS, CHUNK)
    zeros_hbm = jnp.zeros((ZROWS, 16), F32)

    def seg16(z16):
        zp = jnp.concatenate([z16, jnp.zeros((NP - N, 16), F32)], axis=0)
        o = _edge_pass(src2d, dst2d, zp, zeros_hbm)
        return o[:N] + o[NP:NP + N]

    def agg(z):
        wd = z.shape[1]
        ns = -(-wd // 16)
        zq = jnp.pad(z, ((0, 0), (0, ns * 16 - wd)))
        parts = [seg16(zq[:, 16 * k:16 * (k + 1)]) for k in range(ns)]
        return jnp.concatenate(parts, axis=1)[:, :wd]

    deg = seg16(jnp.ones((N, 16), F32))[:, 0] + 1.0
    dinv = lax.rsqrt(deg)
    dcol = dinv[:, None]

    def cgn(x, layers):
        h = x
        for (W, b) in layers:
            z = (h @ W) * dcol
            sagg = agg(z)
            h = jax.nn.relu(dcol * (sagg + z) + b)
        return h

    def fc(x, layers):
        for W, b in layers[:-1]:
            x = jax.nn.relu(x @ W + b)
        W, b = layers[-1]
        return x @ W + b

    p = params
    h = fc(spatial_features, p['spatial_fc'])
    h = h + cgn(h, p['gcn0'])
    h = h + cgn(h, p['gcn1'])
    s_out = cgn(h, p['gcn2'])

    ht = temporal_features[:, 0, :]
    for bp in p['st']:
        sp = cgn(ht, bp['gcn'])
        temp = sp @ bp['conv_w'][:, :, 1].T + bp['conv_b']
        ht = jnp.concatenate([ht, temp], axis=1)
    t_out = jnp.stack([ht[:, 0:12].mean(axis=1), ht[:, 12:24].mean(axis=1)],
                      axis=1)

    e_out = fc(external_features, p['ext_fc'])
    feats = jnp.concatenate([s_out, t_out, e_out], axis=-1)
    W, b = p['out'][0]
    return jax.nn.relu(feats) @ W + b


# speedup vs baseline: 14.2194x; 1.0938x over previous
import functools

import jax
import jax.numpy as jnp
from jax import lax
from jax.experimental import pallas as pl
from jax.experimental.pallas import tpu as pltpu
from jax.experimental.pallas import tpu_sc as plsc

N = 100000
E = 1600000
NP = 102400
DUMMY = N
CHUNK = 128
EP = 1605632
NCHUNKS = EP // CHUNK
CPT_FULL = NCHUNKS // 16
CPT_HALF = NCHUNKS // 32
IBLK = 14
NB_FULL = CPT_FULL // IBLK
NB_HALF = CPT_HALF // IBLK
NBUF = 8
DEPTH = 4
ROWS_PER_TILE = NP // 16
ZROWS = 400
F32 = jnp.float32

_PLANS = {
    1: ([(0, 0, 0)], [(0, 1, 1)]),
    2: ([(0, 0, None)], [(1, 1, None)]),
    3: ([(0, 0, None), (2, 2, 0)], [(1, 1, None), (2, 3, 1)]),
}
_NOUT = {1: 2, 2: 2, 3: 4}


def _make_pass(S):
    plans = _PLANS[S]

    def body(src_hbm, dst_hbm, z_hbm, zero_hbm, out_hbm,
             srcb, dstb, zbuf, *rest):
        rows = list(rest[:NBUF])
        acc = rest[NBUF]
        gsems = list(rest[NBUF + 1:NBUF + 1 + NBUF])
        ssems = list(rest[NBUF + 1 + NBUF:])
        c = lax.axis_index("c")
        s = lax.axis_index("s")
        row0 = s * ROWS_PER_TILE
        pltpu.sync_copy(zero_hbm, zbuf)

        def sweep(reg, outr, half):
            for i in range(ROWS_PER_TILE // ZROWS):
                pltpu.sync_copy(zbuf, acc.at[pl.ds(row0 + i * ZROWS, ZROWS)])
            plsc.subcore_barrier()
            if half is None:
                chunk0 = s * CPT_FULL
                nblocks = NB_FULL
            else:
                chunk0 = half * (NCHUNKS // 2) + s * CPT_HALF
                nblocks = NB_HALF
            sbase = reg * NCHUNKS + chunk0

            def blk(bi, carry):
                pltpu.sync_copy(src_hbm.at[pl.ds(sbase + bi * IBLK, IBLK)],
                                srcb)
                pltpu.sync_copy(dst_hbm.at[pl.ds(chunk0 + bi * IBLK, IBLK)],
                                dstb)
                gh = [None] * IBLK
                sh = [None] * IBLK

                def issue_scatter(k):
                    gh[k].wait()
                    sh[k] = pltpu.async_copy(rows[k % NBUF],
                                             acc.at[dstb.at[k]],
                                             ssems[k % NBUF], add=True)

                for j in range(IBLK):
                    if j >= NBUF:
                        sh[j - NBUF].wait()
                    gh[j] = pltpu.async_copy(z_hbm.at[srcb.at[j]],
                                             rows[j % NBUF], gsems[j % NBUF])
                    if j >= DEPTH:
                        issue_scatter(j - DEPTH)
                for k in range(IBLK - DEPTH, IBLK):
                    issue_scatter(k)
                for k in range(max(0, IBLK - NBUF), IBLK):
                    sh[k].wait()
                return carry

            lax.fori_loop(0, nblocks, blk, 0)
            plsc.subcore_barrier()
            pltpu.sync_copy(acc.at[pl.ds(row0, ROWS_PER_TILE)],
                            out_hbm.at[pl.ds(outr * NP + row0,
                                             ROWS_PER_TILE)])

        for ci in (0, 1):
            @pl.when(c == ci)
            def _run(ci=ci):
                for (reg, outr, half) in plans[ci]:
                    sweep(reg, outr, half)

    return pl.kernel(
        body,
        out_type=jax.ShapeDtypeStruct((_NOUT[S] * NP, 16), F32),
        mesh=plsc.VectorSubcoreMesh(core_axis_name="c", subcore_axis_name="s"),
        compiler_params=pltpu.CompilerParams(use_tc_tiling_on_sc=False),
        scratch_types=[
            pltpu.VMEM((IBLK, CHUNK), jnp.int32),
            pltpu.VMEM((IBLK, CHUNK), jnp.int32),
            pltpu.VMEM((ZROWS, 16), F32),
            *[pltpu.VMEM((CHUNK, 16), F32) for _ in range(NBUF)],
            pltpu.VMEM_SHARED((NP, 16), F32),
            *[pltpu.SemaphoreType.DMA for _ in range(2 * NBUF)],
        ],
    )


_PASS_CACHE = {}


def _get_pass(S):
    if S not in _PASS_CACHE:
        _PASS_CACHE[S] = _make_pass(S)
    return _PASS_CACHE[S]


def kernel(spatial_features, temporal_features, external_features, edges, params):
    src = edges[0]
    dst = edges[1]
    pad = EP - E
    src1 = jnp.concatenate([src, jnp.zeros((pad,), jnp.int32)])
    dst2d = jnp.concatenate(
        [dst, jnp.full((pad,), DUMMY, jnp.int32)]).reshape(NCHUNKS, CHUNK)
    srcr = [(src1 + (r * NP)).reshape(NCHUNKS, CHUNK) for r in range(3)]
    src_flat = {S: jnp.concatenate(srcr[:S], axis=0) for S in (1, 2, 3)}
    zeros_hbm = jnp.zeros((ZROWS, 16), F32)

    def agg(z):
        wd = z.shape[1]
        S = -(-wd // 16)
        zq = jnp.pad(z, ((0, NP - N), (0, S * 16 - wd)))
        z_flat = jnp.concatenate(
            [zq[:, 16 * k:16 * (k + 1)] for k in range(S)], axis=0)
        o = _get_pass(S)(src_flat[S], dst2d, z_flat, zeros_hbm)
        if S == 1:
            parts = [o[:N] + o[NP:NP + N]]
        elif S == 2:
            parts = [o[:N], o[NP:NP + N]]
        else:
            parts = [o[:N], o[NP:NP + N], o[2 * NP:2 * NP + N] + o[3 * NP:3 * NP + N]]
        return jnp.concatenate(parts, axis=1)[:, :wd]

    deg = agg(jnp.ones((N, 16), F32))[:, 0] + 1.0
    dinv = lax.rsqrt(deg)
    dcol = dinv[:, None]

    def fc(x, layers):
        for W, b in layers[:-1]:
            x = jax.nn.relu(x @ W + b)
        W, b = layers[-1]
        return x @ W + b

    p = params
    sp_layers = list(p['gcn0']) + list(p['gcn1']) + list(p['gcn2'])
    t_layers = [lyr for bp in p['st'] for lyr in bp['gcn']]

    h = fc(spatial_features, p['spatial_fc'])
    xs = h
    ht = temporal_features[:, 0, :]
    xt = ht
    for r in range(13):
        Ws, bs = sp_layers[r]
        zs = (xs @ Ws) * dcol
        if r < 12:
            Wt, bt = t_layers[r]
            zt = (xt @ Wt) * dcol
            sagg = agg(jnp.concatenate([zs, zt], axis=1))
            ss = sagg[:, :zs.shape[1]]
            st = sagg[:, zs.shape[1]:]
            xt = jax.nn.relu(dcol * (st + zt) + bt)
        else:
            ss = agg(zs)
        xs = jax.nn.relu(dcol * (ss + zs) + bs)
        if r == 3:
            h = h + xs
            xs = h
        elif r == 7:
            h = h + xs
            xs = h
        if r < 12 and r % 3 == 2:
            bp = p['st'][r // 3]
            temp = xt @ bp['conv_w'][:, :, 1].T + bp['conv_b']
            ht = jnp.concatenate([ht, temp], axis=1)
            xt = ht

    s_out = xs
    t_out = jnp.stack([ht[:, 0:12].mean(axis=1), ht[:, 12:24].mean(axis=1)],
                      axis=1)
    e_out = fc(external_features, p['ext_fc'])
    feats = jnp.concatenate([s_out, t_out, e_out], axis=-1)
    W, b = p['out'][0]
    return jax.nn.relu(feats) @ W + b


# speedup vs baseline: 14.4021x; 1.0128x over previous
import functools

import jax
import jax.numpy as jnp
from jax import lax
from jax.experimental import pallas as pl
from jax.experimental.pallas import tpu as pltpu
from jax.experimental.pallas import tpu_sc as plsc

N = 100000
E = 1600000
NP = 100352
DUMMY = N
CHUNK = 128
EP = 1605632
NCHUNKS = EP // CHUNK
CPT_FULL = NCHUNKS // 16
CPT_HALF = NCHUNKS // 32
IBLK = 14
NB_FULL = CPT_FULL // IBLK
NB_HALF = CPT_HALF // IBLK
NBUF = 10
DEPTH = 5
ROWS_PER_TILE = NP // 16
ZROWS = 392
F32 = jnp.float32

_PLANS = {
    1: ([(0, 0, 0)], [(0, 1, 1)]),
    2: ([(0, 0, None)], [(1, 1, None)]),
    3: ([(0, 0, None), (2, 2, 0)], [(1, 1, None), (2, 3, 1)]),
}
_NOUT = {1: 2, 2: 2, 3: 4}


def _make_pass(S):
    plans = _PLANS[S]

    def body(src_hbm, dst_hbm, z_hbm, zero_hbm, out_hbm,
             srcb, dstb, zbuf, *rest):
        rows = list(rest[:NBUF])
        acc = rest[NBUF]
        gsems = list(rest[NBUF + 1:NBUF + 1 + NBUF])
        ssems = list(rest[NBUF + 1 + NBUF:])
        c = lax.axis_index("c")
        s = lax.axis_index("s")
        row0 = s * ROWS_PER_TILE
        pltpu.sync_copy(zero_hbm, zbuf)

        def sweep(reg, outr, half):
            for i in range(ROWS_PER_TILE // ZROWS):
                pltpu.sync_copy(zbuf, acc.at[pl.ds(row0 + i * ZROWS, ZROWS)])
            plsc.subcore_barrier()
            if half is None:
                chunk0 = s * CPT_FULL
                nblocks = NB_FULL
            else:
                chunk0 = half * (NCHUNKS // 2) + s * CPT_HALF
                nblocks = NB_HALF
            sbase = reg * NCHUNKS + chunk0

            def blk(bi, carry):
                pltpu.sync_copy(src_hbm.at[pl.ds(sbase + bi * IBLK, IBLK)],
                                srcb)
                pltpu.sync_copy(dst_hbm.at[pl.ds(chunk0 + bi * IBLK, IBLK)],
                                dstb)
                gh = [None] * IBLK
                sh = [None] * IBLK

                def issue_scatter(k):
                    gh[k].wait()
                    sh[k] = pltpu.async_copy(rows[k % NBUF],
                                             acc.at[dstb.at[k]],
                                             ssems[k % NBUF], add=True)

                for j in range(IBLK):
                    if j >= NBUF:
                        sh[j - NBUF].wait()
                    gh[j] = pltpu.async_copy(z_hbm.at[srcb.at[j]],
                                             rows[j % NBUF], gsems[j % NBUF])
                    if j >= DEPTH:
                        issue_scatter(j - DEPTH)
                for k in range(IBLK - DEPTH, IBLK):
                    issue_scatter(k)
                for k in range(max(0, IBLK - NBUF), IBLK):
                    sh[k].wait()
                return carry

            lax.fori_loop(0, nblocks, blk, 0)
            plsc.subcore_barrier()
            pltpu.sync_copy(acc.at[pl.ds(row0, ROWS_PER_TILE)],
                            out_hbm.at[pl.ds(outr * NP + row0,
                                             ROWS_PER_TILE)])

        for ci in (0, 1):
            @pl.when(c == ci)
            def _run(ci=ci):
                for (reg, outr, half) in plans[ci]:
                    sweep(reg, outr, half)

    return pl.kernel(
        body,
        out_type=jax.ShapeDtypeStruct((_NOUT[S] * NP, 16), F32),
        mesh=plsc.VectorSubcoreMesh(core_axis_name="c", subcore_axis_name="s"),
        compiler_params=pltpu.CompilerParams(use_tc_tiling_on_sc=False),
        scratch_types=[
            pltpu.VMEM((IBLK, CHUNK), jnp.int32),
            pltpu.VMEM((IBLK, CHUNK), jnp.int32),
            pltpu.VMEM((ZROWS, 16), F32),
            *[pltpu.VMEM((CHUNK, 16), F32) for _ in range(NBUF)],
            pltpu.VMEM_SHARED((NP, 16), F32),
            *[pltpu.SemaphoreType.DMA for _ in range(2 * NBUF)],
        ],
    )


_PASS_CACHE = {}


def _get_pass(S):
    if S not in _PASS_CACHE:
        _PASS_CACHE[S] = _make_pass(S)
    return _PASS_CACHE[S]


def kernel(spatial_features, temporal_features, external_features, edges, params):
    src = edges[0]
    dst = edges[1]
    pad = EP - E
    src1 = jnp.concatenate([src, jnp.zeros((pad,), jnp.int32)])
    dst2d = jnp.concatenate(
        [dst, jnp.full((pad,), DUMMY, jnp.int32)]).reshape(NCHUNKS, CHUNK)
    srcr = [(src1 + (r * NP)).reshape(NCHUNKS, CHUNK) for r in range(3)]
    src_flat = {S: jnp.concatenate(srcr[:S], axis=0) for S in (1, 2, 3)}
    zeros_hbm = jnp.zeros((ZROWS, 16), F32)

    def agg(z):
        wd = z.shape[1]
        S = -(-wd // 16)
        zq = jnp.pad(z, ((0, NP - N), (0, S * 16 - wd)))
        z_flat = jnp.concatenate(
            [zq[:, 16 * k:16 * (k + 1)] for k in range(S)], axis=0)
        o = _get_pass(S)(src_flat[S], dst2d, z_flat, zeros_hbm)
        if S == 1:
            parts = [o[:N] + o[NP:NP + N]]
        elif S == 2:
            parts = [o[:N], o[NP:NP + N]]
        else:
            parts = [o[:N], o[NP:NP + N], o[2 * NP:2 * NP + N] + o[3 * NP:3 * NP + N]]
        return jnp.concatenate(parts, axis=1)[:, :wd]

    deg = agg(jnp.ones((N, 16), F32))[:, 0] + 1.0
    dinv = lax.rsqrt(deg)
    dcol = dinv[:, None]

    def fc(x, layers):
        for W, b in layers[:-1]:
            x = jax.nn.relu(x @ W + b)
        W, b = layers[-1]
        return x @ W + b

    p = params
    sp_layers = list(p['gcn0']) + list(p['gcn1']) + list(p['gcn2'])
    t_layers = [lyr for bp in p['st'] for lyr in bp['gcn']]

    h = fc(spatial_features, p['spatial_fc'])
    xs = h
    ht = temporal_features[:, 0, :]
    xt = ht
    for r in range(13):
        Ws, bs = sp_layers[r]
        zs = (xs @ Ws) * dcol
        if r < 12:
            Wt, bt = t_layers[r]
            zt = (xt @ Wt) * dcol
            sagg = agg(jnp.concatenate([zs, zt], axis=1))
            ss = sagg[:, :zs.shape[1]]
            st = sagg[:, zs.shape[1]:]
            xt = jax.nn.relu(dcol * (st + zt) + bt)
        else:
            ss = agg(zs)
        xs = jax.nn.relu(dcol * (ss + zs) + bs)
        if r == 3:
            h = h + xs
            xs = h
        elif r == 7:
            h = h + xs
            xs = h
        if r < 12 and r % 3 == 2:
            bp = p['st'][r // 3]
            temp = xt @ bp['conv_w'][:, :, 1].T + bp['conv_b']
            ht = jnp.concatenate([ht, temp], axis=1)
            xt = ht

    s_out = xs
    t_out = jnp.stack([ht[:, 0:12].mean(axis=1), ht[:, 12:24].mean(axis=1)],
                      axis=1)
    e_out = fc(external_features, p['ext_fc'])
    feats = jnp.concatenate([s_out, t_out, e_out], axis=-1)
    W, b = p['out'][0]
    return jax.nn.relu(feats) @ W + b
